# Initial kernel scaffold; baseline (speedup 1.0000x reference)
import jax
import jax.numpy as jnp
from jax.experimental import pallas as pl


def kernel(x, pe):
    raise NotImplementedError("write your pallas kernel here")


# speedup vs baseline: 1.7166x; 1.7166x over previous
import jax
import jax.numpy as jnp
from jax.experimental import pallas as pl
from jax.experimental.pallas import tpu as pltpu

_BLK_S = 256


def _add_pe_kernel(x_ref, pe_ref, o_ref):
    o_ref[...] = x_ref[...] + pe_ref[...][jnp.newaxis, :, :]


def kernel(x, pe):
    B, S, D = x.shape
    pe_rows = pe[:S]
    grid = (S // _BLK_S,)
    return pl.pallas_call(
        _add_pe_kernel,
        grid=grid,
        in_specs=[
            pl.BlockSpec((B, _BLK_S, D), lambda s: (0, s, 0)),
            pl.BlockSpec((_BLK_S, D), lambda s: (s, 0)),
        ],
        out_specs=pl.BlockSpec((B, _BLK_S, D), lambda s: (0, s, 0)),
        out_shape=jax.ShapeDtypeStruct((B, S, D), x.dtype),
        compiler_params=pltpu.CompilerParams(
            dimension_semantics=("parallel",),
        ),
    )(x, pe_rows)


# speedup vs baseline: 1.7253x; 1.0051x over previous
import jax
import jax.numpy as jnp
from jax.experimental import pallas as pl
from jax.experimental.pallas import tpu as pltpu

_BLK_S = 512


def _add_pe_kernel(x_ref, pe_ref, o_ref):
    o_ref[...] = x_ref[...] + pe_ref[...][jnp.newaxis, :, :]


def kernel(x, pe):
    B, S, D = x.shape
    pe_rows = pe[:S]
    grid = (S // _BLK_S,)
    return pl.pallas_call(
        _add_pe_kernel,
        grid=grid,
        in_specs=[
            pl.BlockSpec((B, _BLK_S, D), lambda s: (0, s, 0)),
            pl.BlockSpec((_BLK_S, D), lambda s: (s, 0)),
        ],
        out_specs=pl.BlockSpec((B, _BLK_S, D), lambda s: (0, s, 0)),
        out_shape=jax.ShapeDtypeStruct((B, S, D), x.dtype),
        compiler_params=pltpu.CompilerParams(
            dimension_semantics=("parallel",),
        ),
    )(x, pe_rows)
